# baseline (speedup 1.0000x reference)
import functools

import jax
import jax.numpy as jnp
from jax import lax
from jax.experimental import pallas as pl
from jax.experimental.pallas import tpu as pltpu
from jax.experimental.pallas import tpu_sc as plsc

_NC = 2
_NS = 16
_L = 16
_CHUNK = 128
_FILLS = 1024
_FIX_ROUNDS = 3

_PARAMS = pltpu.CompilerParams(
    needs_layout_passes=False, use_tc_tiling_on_sc=False)


def _iota16():
    return lax.broadcasted_iota(jnp.int32, (_L,), 0)


def _splat(x):
    return jnp.full((_L,), x, jnp.int32)


def _make_stage1(M, D, B, S):
    wpt = B // _NS
    wk = wpt // _CHUNK
    spw = S // (_NC * _NS)
    sk = spw // _CHUNK
    span = ((M + _NS * _FILLS - 1) // (_NS * _FILLS)) * _FILLS
    dummy = _NS * span
    table_n = dummy + _L

    mesh = plsc.VectorSubcoreMesh(core_axis_name="c", subcore_axis_name="s")

    @functools.partial(
        pl.kernel,
        mesh=mesh,
        out_type=jax.ShapeDtypeStruct((_NS * (((M + _NS * _FILLS - 1)
                  // (_NS * _FILLS)) * _FILLS) + _L,), jnp.int32),
        scratch_types=[
            pltpu.VMEM_SHARED((table_n,), jnp.int32),
            pltpu.VMEM((_FILLS,), jnp.int32),
            pltpu.VMEM((wk, _CHUNK), jnp.int32),
            pltpu.VMEM((wk, _CHUNK), jnp.int32),
            pltpu.VMEM((_CHUNK,), jnp.int32),
            pltpu.VMEM((_CHUNK,), jnp.int32),
        ],
        compiler_params=_PARAMS,
    )
    def stage1(widx_h, table_h,
               table, fillv, widx_v, jval_v, tvrow, fixrow):
        c = lax.axis_index("c")
        s = lax.axis_index("s")
        wid = c * _NS + s
        i16 = _iota16()

        for q in range(_FILLS // _L):
            fillv[pl.ds(q * _L, _L)] = jnp.full((_L,), -1, jnp.int32)
        base = s * span

        def memset_body(q, _):
            pltpu.sync_copy(fillv, table.at[pl.ds(base + q * _FILLS, _FILLS)])
            return _
        lax.fori_loop(0, span // _FILLS, memset_body, None)

        pltpu.sync_copy(widx_h.at[s], widx_v)
        jbase = s * wpt
        for k in range(wk):
            for v in range(_CHUNK // _L):
                jval_v[k, pl.ds(v * _L, _L)] = i16 + (jbase + k * _CHUNK + v * _L)

        plsc.subcore_barrier()

        for k in range(wk):
            pltpu.sync_copy(jval_v.at[k], table.at[widx_v.at[k]])

        for _r in range(_FIX_ROUNDS):
            plsc.subcore_barrier()

            def fix_body(k, _):
                pltpu.sync_copy(table.at[widx_v.at[k]], tvrow)
                for v in range(_CHUNK // _L):
                    cidx = i16 + v * _L
                    jv = plsc.load_gather(jval_v, [_splat(k), cidx])
                    wv = plsc.load_gather(widx_v, [_splat(k), cidx])
                    tvv = tvrow[pl.ds(v * _L, _L)]
                    fixrow[pl.ds(v * _L, _L)] = jnp.where(tvv < jv, wv, dummy)
                pltpu.sync_copy(jval_v.at[k], table.at[fixrow])
                return _
            lax.fori_loop(0, wk, fix_body, None)

        plsc.subcore_barrier()

        @pl.when(c == 0)
        def _():
            pltpu.sync_copy(table.at[pl.ds(base, span)],
                            table_h.at[pl.ds(base, span)])

    return stage1


def _make_stage2(M, D, B, S):
    spw = S // (_NC * _NS)
    sk = spw // _CHUNK
    mesh = plsc.VectorSubcoreMesh(core_axis_name="c", subcore_axis_name="s")

    @functools.partial(
        pl.kernel,
        mesh=mesh,
        out_type=jax.ShapeDtypeStruct((S, D), jnp.float32),
        scratch_types=(
            [pltpu.VMEM((_CHUNK,), jnp.int32)] * (3 * 4)
            + [pltpu.VMEM((_CHUNK,), jnp.float32)] * (4 * 4)
            + [pltpu.VMEM((_CHUNK, D), jnp.float32)] * (2 * 4)
            + [pltpu.SemaphoreType.DMA] * 5
        ),
        compiler_params=_PARAMS,
    )
    def stage2(mem, mimp, feats, fimp, table_h, sidx_h, out, *scr):
        c = lax.axis_index("c")
        s = lax.axis_index("s")
        wid = c * _NS + s
        i16 = _iota16()
        srow = scr[0:4]
        jwrow = scr[4:8]
        idxb = scr[8:12]
        impa = scr[12:16]
        impb = scr[16:20]
        w2 = scr[20:24]
        hv = scr[24:28]
        mema = scr[28:32]
        featb = scr[32:36]
        sem_m, sem_i, sem_jw, sem_b, sem_j = scr[36:41]

        for k in range(sk):
            pltpu.sync_copy(sidx_h.at[wid, k], srow[k])
        cpm = [pltpu.async_copy(mem.at[srow[k]], mema[k], sem_m)
               for k in range(sk)]
        cpi = [pltpu.async_copy(mimp.at[srow[k]], impa[k], sem_i)
               for k in range(sk)]
        cpjw = [pltpu.async_copy(table_h.at[srow[k]], jwrow[k], sem_jw)
                for k in range(sk)]
        for cp in cpjw:
            cp.wait()
        for k in range(sk):
            for v in range(_CHUNK // _L):
                sl = pl.ds(v * _L, _L)
                jw = jwrow[k][sl]
                idxb[k][sl] = jnp.where(jw >= 0, jw, 0)
        cpb = [pltpu.async_copy(feats.at[idxb[k]], featb[k], sem_b)
               for k in range(sk)]
        cpj = [pltpu.async_copy(fimp.at[idxb[k]], impb[k], sem_j)
               for k in range(sk)]
        for cp in cpi:
            cp.wait()
        for cp in cpj:
            cp.wait()
        for k in range(sk):
            for v in range(_CHUNK // _L):
                sl = pl.ds(v * _L, _L)
                hit = jwrow[k][sl] >= 0
                w2[k][sl] = jnp.where(hit, 0.0, impa[k][sl])
                hv[k][sl] = jnp.where(hit, impb[k][sl], 0.0)
        for cp in cpm:
            cp.wait()
        for cp in cpb:
            cp.wait()
        for k in range(sk):
            w2k, hvk, mek, fbk = w2[k], hv[k], mema[k], featb[k]

            def row_body(i, si0, w2k=w2k, hvk=hvk, mek=mek, fbk=fbk):
                for u in range(4):
                    si = si0 + u
                    w16 = plsc.load_gather(w2k, [si])
                    h16 = plsc.load_gather(hvk, [si])
                    for ccol in range(D // _L):
                        cidx = i16 + ccol * _L
                        a = plsc.load_gather(mek, [si, cidx])
                        b = plsc.load_gather(fbk, [si, cidx])
                        plsc.store_scatter(mek, [si, cidx],
                                           a * w16 + b * h16)
                return si0 + 4
            lax.fori_loop(0, _CHUNK // 4, row_body,
                          jnp.zeros((_L,), jnp.int32))
            pltpu.sync_copy(mema[k], out.at[pl.ds(wid * spw + k * _CHUNK,
                                                  _CHUNK)])

    return stage2


def kernel(memory_features, memory_importance, features, importance,
           write_idx, sample_idx):
    M, D = memory_features.shape
    B = write_idx.shape[0]
    S = sample_idx.shape[0]
    widx3 = write_idx.reshape(_NS, B // (_NS * _CHUNK), _CHUNK)
    sidx3 = sample_idx.reshape(_NC * _NS, S // (_NC * _NS * _CHUNK), _CHUNK)
    table_h = _make_stage1(M, D, B, S)(widx3)
    return _make_stage2(M, D, B, S)(
        memory_features, memory_importance, features, importance,
        table_h, sidx3)


# speedup vs baseline: 1.0120x; 1.0120x over previous
import functools

import jax
import jax.numpy as jnp
from jax import lax
from jax.experimental import pallas as pl
from jax.experimental.pallas import tpu as pltpu
from jax.experimental.pallas import tpu_sc as plsc

_NC = 2
_NS = 16
_L = 16
_CHUNK = 128
_FILLS = 1024
_FIX_ROUNDS = 3

_PARAMS = pltpu.CompilerParams(
    needs_layout_passes=False, use_tc_tiling_on_sc=False)


def _iota16():
    return lax.broadcasted_iota(jnp.int32, (_L,), 0)


def _splat(x):
    return jnp.full((_L,), x, jnp.int32)


def _make_stage1(M, D, B, S):
    wpt = B // _NS
    wk = wpt // _CHUNK
    spw = S // (_NC * _NS)
    sk = spw // _CHUNK
    span = ((M + _NS * _FILLS - 1) // (_NS * _FILLS)) * _FILLS
    dummy = _NS * span
    table_n = dummy + _L

    mesh = plsc.VectorSubcoreMesh(core_axis_name="c", subcore_axis_name="s")

    @functools.partial(
        pl.kernel,
        mesh=mesh,
        out_type=(jax.ShapeDtypeStruct((S, D), jnp.float32),
                  jax.ShapeDtypeStruct((S, D), jnp.float32)),
        scratch_types=[
            pltpu.VMEM_SHARED((table_n,), jnp.int32),
            pltpu.VMEM((_FILLS,), jnp.int32),
            pltpu.VMEM((wk, _CHUNK), jnp.int32),
            pltpu.VMEM((wk, _CHUNK), jnp.int32),
            pltpu.VMEM((_CHUNK,), jnp.int32),
            pltpu.VMEM((_CHUNK,), jnp.int32),
            pltpu.VMEM((_CHUNK,), jnp.int32),
            pltpu.VMEM((_CHUNK,), jnp.int32),
            pltpu.VMEM((_CHUNK,), jnp.int32),
            pltpu.VMEM((_CHUNK,), jnp.float32),
            pltpu.VMEM((_CHUNK,), jnp.float32),
            pltpu.VMEM((_CHUNK,), jnp.float32),
            pltpu.VMEM((_CHUNK,), jnp.float32),
            pltpu.VMEM((_CHUNK, D), jnp.float32),
            pltpu.VMEM((_CHUNK, D), jnp.float32),
            pltpu.VMEM((_CHUNK, D), jnp.float32),
        ],
        compiler_params=_PARAMS,
    )
    def stage1(mimp, feats, fimp, widx_h, sidx_h, hit_h, wexp_h,
               table, fillv, widx_v, jval_v, tvrow, fixrow,
               srow, jwrow, idxbrow, impa, impb, w2row, hvrow,
               featb, hrows, wbuf):
        c = lax.axis_index("c")
        s = lax.axis_index("s")
        wid = c * _NS + s
        i16 = _iota16()

        for q in range(_FILLS // _L):
            fillv[pl.ds(q * _L, _L)] = jnp.full((_L,), -1, jnp.int32)
        base = s * span

        def memset_body(q, _):
            pltpu.sync_copy(fillv, table.at[pl.ds(base + q * _FILLS, _FILLS)])
            return _
        lax.fori_loop(0, span // _FILLS, memset_body, None)

        pltpu.sync_copy(widx_h.at[s], widx_v)
        jbase = s * wpt
        for k in range(wk):
            for v in range(_CHUNK // _L):
                jval_v[k, pl.ds(v * _L, _L)] = i16 + (jbase + k * _CHUNK + v * _L)

        plsc.subcore_barrier()

        for k in range(wk):
            pltpu.sync_copy(jval_v.at[k], table.at[widx_v.at[k]])

        for _r in range(_FIX_ROUNDS):
            plsc.subcore_barrier()

            def fix_body(k, _):
                pltpu.sync_copy(table.at[widx_v.at[k]], tvrow)
                for v in range(_CHUNK // _L):
                    cidx = i16 + v * _L
                    jv = plsc.load_gather(jval_v, [_splat(k), cidx])
                    wv = plsc.load_gather(widx_v, [_splat(k), cidx])
                    tvv = tvrow[pl.ds(v * _L, _L)]
                    fixrow[pl.ds(v * _L, _L)] = jnp.where(tvv < jv, wv, dummy)
                pltpu.sync_copy(jval_v.at[k], table.at[fixrow])
                return _
            lax.fori_loop(0, wk, fix_body, None)

        plsc.subcore_barrier()

        def chunk_body(k, _):
            pltpu.sync_copy(sidx_h.at[wid, k], srow)
            pltpu.sync_copy(table.at[srow], jwrow)
            pltpu.sync_copy(mimp.at[srow], impa)
            for v in range(_CHUNK // _L):
                sl = pl.ds(v * _L, _L)
                jw = jwrow[sl]
                idxbrow[sl] = jnp.where(jw >= 0, jw, 0)
            pltpu.sync_copy(feats.at[idxbrow], featb)
            pltpu.sync_copy(fimp.at[idxbrow], impb)
            for v in range(_CHUNK // _L):
                sl = pl.ds(v * _L, _L)
                hit = jwrow[sl] >= 0
                w2row[sl] = jnp.where(hit, 0.0, impa[sl])
                hvrow[sl] = jnp.where(hit, impb[sl], 0.0)

            def row_body(i, _):
                si = _splat(i)
                w16 = plsc.load_gather(w2row, [si])
                h16 = plsc.load_gather(hvrow, [si])
                for ccol in range(D // _L):
                    cidx = i16 + ccol * _L
                    b = plsc.load_gather(featb, [si, cidx])
                    plsc.store_scatter(hrows, [si, cidx], b * h16)
                    plsc.store_scatter(wbuf, [si, cidx], w16)
                return _
            lax.fori_loop(0, _CHUNK, row_body, None)

            rbase = wid * spw + k * _CHUNK
            pltpu.sync_copy(hrows, hit_h.at[pl.ds(rbase, _CHUNK)])
            pltpu.sync_copy(wbuf, wexp_h.at[pl.ds(rbase, _CHUNK)])
            return _
        lax.fori_loop(0, sk, chunk_body, None)

    return stage1


def _make_stage2(M, D, S):
    spw = S // (_NC * _NS)
    sk = spw // _CHUNK
    mesh = plsc.VectorSubcoreMesh(core_axis_name="c", subcore_axis_name="s")

    @functools.partial(
        pl.kernel,
        mesh=mesh,
        out_type=jax.ShapeDtypeStruct((S, D), jnp.float32),
        scratch_types=[
            pltpu.VMEM((_CHUNK,), jnp.int32),
            pltpu.VMEM((_CHUNK, D), jnp.float32),
            pltpu.VMEM((_CHUNK, D), jnp.float32),
            pltpu.VMEM((_CHUNK, D), jnp.float32),
            pltpu.SemaphoreType.DMA,
        ],
        compiler_params=_PARAMS,
    )
    def stage2(mem, sidx_h, hit_h, wexp_h, out, srow, mema, hr, wx, sem):
        c = lax.axis_index("c")
        s = lax.axis_index("s")
        wid = c * _NS + s

        def chunk_body(k, _):
            rbase = wid * spw + k * _CHUNK
            pltpu.sync_copy(sidx_h.at[wid, k], srow)
            cpa = pltpu.async_copy(mem.at[srow], mema, sem)
            cph = pltpu.async_copy(hit_h.at[pl.ds(rbase, _CHUNK)], hr, sem)
            cpw = pltpu.async_copy(wexp_h.at[pl.ds(rbase, _CHUNK)], wx, sem)
            cpa.wait()
            cph.wait()
            cpw.wait()
            for r in range(_CHUNK):
                for ccol in range(0, D, _L):
                    sl = pl.ds(ccol, _L)
                    mema[r, sl] = mema[r, sl] * wx[r, sl] + hr[r, sl]
            pltpu.sync_copy(mema, out.at[pl.ds(rbase, _CHUNK)])
            return _
        lax.fori_loop(0, sk, chunk_body, None)

    return stage2


def kernel(memory_features, memory_importance, features, importance,
           write_idx, sample_idx):
    M, D = memory_features.shape
    B = write_idx.shape[0]
    S = sample_idx.shape[0]
    widx3 = write_idx.reshape(_NS, B // (_NS * _CHUNK), _CHUNK)
    sidx3 = sample_idx.reshape(_NC * _NS, S // (_NC * _NS * _CHUNK), _CHUNK)
    hitrows, wexp = _make_stage1(M, D, B, S)(
        memory_importance, features, importance, widx3, sidx3)
    return _make_stage2(M, D, S)(memory_features, sidx3, hitrows, wexp)
